# baseline (speedup 1.0000x reference)
import functools

import jax
import jax.numpy as jnp
from jax import lax
from jax.experimental import pallas as pl
from jax.experimental.pallas import tpu as pltpu
from jax.experimental.pallas import tpu_sc as plsc

NC, NS, L = 2, 16, 16
NW = NC * NS
D = 64
SCALE = 8.0
CHUNK = 256
SGRP = 12
DGRP = 4
SROWS = SGRP * L
DROWS = DGRP * L
NBUF = 4
TCG = 32


@functools.partial(jax.jit, static_argnames=("B",))
def _sc_gather(idx_flat, table, B):
    b_per_w = B // NW
    n_chunks = b_per_w // CHUNK
    assert n_chunks % NBUF == 0 and n_chunks >= 2 * NBUF
    mesh = plsc.VectorSubcoreMesh(core_axis_name="c", subcore_axis_name="s")

    @functools.partial(
        pl.kernel,
        out_type=jax.ShapeDtypeStruct((B, D), jnp.float32),
        mesh=mesh,
        scratch_types=[
            pltpu.VMEM((b_per_w,), jnp.int32),
            pltpu.VMEM((NBUF, SROWS, D), jnp.float32),
            pltpu.VMEM_SHARED((NS, 2, SROWS, D), jnp.float32),
            pltpu.VMEM_SHARED((NS, NBUF, DROWS, D), jnp.float32),
            pltpu.SemaphoreType.DMA((NBUF,)),
            pltpu.SemaphoreType.DMA((NBUF,)),
            pltpu.SemaphoreType.DMA((2,)),
            pltpu.SemaphoreType.DMA((NBUF,)),
        ],
        compiler_params=pltpu.CompilerParams(use_tc_tiling_on_sc=False),
    )
    def k(idx_hbm, table_hbm, out_hbm, idx_v, rows_v, swin, dwin,
          sem_g, sem_d, sem_so, sem_do):
        sid = lax.axis_index("s")
        wid = sid * NC + lax.axis_index("c")
        base = wid * b_per_w

        pltpu.sync_copy(
            idx_hbm.at[pl.ds(pl.multiple_of(base, 256), b_per_w)], idx_v)

        def fire_stream(c, b):
            pltpu.async_copy(
                table_hbm.at[idx_v.at[pl.ds(c * CHUNK, SROWS)]],
                rows_v.at[b],
                sem_g.at[b],
            )

        def drain_stream(c, b):
            pltpu.make_async_copy(
                table_hbm.at[idx_v.at[pl.ds(c * CHUNK, SROWS)]],
                rows_v.at[b],
                sem_g.at[b],
            ).wait()

        def fire_dma_rows(c, b):
            def grp(j, carry):
                g16 = idx_v[pl.ds(c * CHUNK + SROWS + j * L, L)]
                for r in range(L):
                    pltpu.async_copy(
                        table_hbm.at[pl.ds(g16[r], 1)],
                        dwin.at[sid, b, pl.ds(j * L + r, 1)],
                        sem_d.at[b],
                    )
                return carry

            lax.fori_loop(0, DGRP, grp, 0, unroll=False)

        def drain_dma_rows(b):
            def grp(j, carry):
                for r in range(L):
                    pltpu.make_async_copy(
                        table_hbm.at[pl.ds(0, 1)],
                        dwin.at[sid, b, pl.ds(0, 1)],
                        sem_d.at[b],
                    ).wait()
                return carry

            lax.fori_loop(0, DGRP, grp, 0, unroll=False)

        def wait_sflush(w2):
            pltpu.make_async_copy(
                swin.at[sid, w2], out_hbm.at[pl.ds(0, SROWS)], sem_so.at[w2]
            ).wait()

        def wait_dflush(b):
            pltpu.make_async_copy(
                dwin.at[sid, b], out_hbm.at[pl.ds(0, DROWS)], sem_do.at[b]
            ).wait()

        def compute(c, b):
            def grp_body(gg, carry):
                g16 = idx_v[pl.ds(c * CHUNK + gg * L, L)]
                m16 = jnp.where(g16 != 0, SCALE, 0.0).astype(jnp.float32)
                for r in range(L):
                    m = m16.at[jnp.full((L,), r, jnp.int32)].get(
                        mode="promise_in_bounds")
                    row = gg * L + r
                    for kk in range(D // L):
                        v = rows_v[b, row, pl.ds(kk * L, L)]
                        rows_v[b, row, pl.ds(kk * L, L)] = v * m
                return carry

            lax.fori_loop(0, SGRP, grp_body, 0, unroll=False)

        for c in range(NBUF - 1):
            fire_stream(c, c)
            fire_dma_rows(c, c)

        def outer_body(g, carry):
            for b in range(NBUF):
                c = g * NBUF + b
                w2 = b % 2
                drain_stream(c, b)
                compute(c, b)

                @pl.when(c >= 2)
                def _():
                    wait_sflush(w2)

                pltpu.sync_copy(rows_v.at[b], swin.at[sid, w2])
                pltpu.async_copy(
                    swin.at[sid, w2],
                    out_hbm.at[pl.ds(base + c * CHUNK, SROWS)],
                    sem_so.at[w2],
                )
                drain_dma_rows(b)
                pltpu.async_copy(
                    dwin.at[sid, b],
                    out_hbm.at[pl.ds(base + c * CHUNK + SROWS, DROWS)],
                    sem_do.at[b],
                )
                bp = (b + NBUF - 1) % NBUF

                @pl.when(c + NBUF - 1 < n_chunks)
                def _():
                    fire_stream(c + NBUF - 1, bp)

                    @pl.when(c >= 1)
                    def _():
                        wait_dflush(bp)

                    fire_dma_rows(c + NBUF - 1, bp)

            return carry

        lax.fori_loop(0, n_chunks // NBUF, outer_body, 0, unroll=False)

        for c in range(n_chunks - 2, n_chunks):
            wait_sflush(c % 2)
        for c in range(n_chunks - NBUF, n_chunks):
            wait_dflush(c % NBUF)

    return k(idx_flat, table)


def _tc_stripe_body(idx_ref, rows_ref, o_ref):
    m = jnp.where(idx_ref[...] != 0, SCALE, 0.0).astype(jnp.float32)
    o_ref[...] = rows_ref[...] * m


@functools.partial(jax.jit, static_argnames=("B",))
def _tc_scale_stripes(idx_flat, gathered, B):
    n_win = B // CHUNK
    i3 = idx_flat.reshape(n_win, CHUNK // DROWS, DROWS, 1)
    g4 = gathered.reshape(n_win, CHUNK // DROWS, DROWS, D)
    stripe = CHUNK // DROWS - 1
    out = pl.pallas_call(
        _tc_stripe_body,
        grid=(n_win // TCG,),
        in_specs=[
            pl.BlockSpec((TCG, 1, DROWS, 1), lambda i: (i, stripe, 0, 0)),
            pl.BlockSpec((TCG, 1, DROWS, D), lambda i: (i, stripe, 0, 0)),
        ],
        out_specs=pl.BlockSpec(
            (TCG, 1, DROWS, D), lambda i: (i, stripe, 0, 0)),
        out_shape=jax.ShapeDtypeStruct((n_win, CHUNK // DROWS, DROWS, D),
                                       jnp.float32),
        input_output_aliases={1: 0},
    )(i3, g4)
    return out.reshape(B, D)


def kernel(inputs, shared_weights):
    B = inputs.size
    idx_flat = inputs.reshape(B).astype(jnp.int32)
    gathered = _sc_gather(idx_flat, shared_weights, B)
    out = _tc_scale_stripes(idx_flat, gathered, B)
    return out.reshape(inputs.shape + (D,))


# speedup vs baseline: 1.3427x; 1.3427x over previous
import functools

import jax
import jax.numpy as jnp
from jax import lax
from jax.experimental import pallas as pl
from jax.experimental.pallas import tpu as pltpu
from jax.experimental.pallas import tpu_sc as plsc

NC, NS, L = 2, 16, 16
NW = NC * NS
D = 64
SCALE = 8.0
SUB = 256
CHUNK = 256
NBUF = 4


@functools.partial(jax.jit, static_argnames=("B",))
def _sc_lookup(idx_flat, table, B):
    b_per_w = B // NW
    n_chunks = b_per_w // CHUNK
    assert n_chunks % NBUF == 0 and n_chunks >= 2 * NBUF
    mesh = plsc.VectorSubcoreMesh(core_axis_name="c", subcore_axis_name="s")

    @functools.partial(
        pl.kernel,
        out_type=jax.ShapeDtypeStruct((B, D), jnp.float32),
        mesh=mesh,
        scratch_types=[
            pltpu.VMEM((b_per_w,), jnp.int32),
            pltpu.VMEM((NBUF, CHUNK, D), jnp.float32),
            pltpu.SemaphoreType.DMA((NBUF,)),
            pltpu.SemaphoreType.DMA((NBUF,)),
        ],
        compiler_params=pltpu.CompilerParams(use_tc_tiling_on_sc=False),
    )
    def k(idx_hbm, table_hbm, out_hbm, idx_v, rows_v, sem_g, sem_o):
        wid = lax.axis_index("s") * NC + lax.axis_index("c")
        base = wid * b_per_w

        pltpu.sync_copy(idx_hbm.at[pl.ds(base, b_per_w)], idx_v)

        def stage_and_fire(c, b):
            def fire(j, carry):
                iv = idx_v[pl.ds(c * CHUNK + j * L, L)]
                pltpu.async_copy(
                    table_hbm.at[iv],
                    rows_v.at[b, pl.ds(j * L, L)],
                    sem_g.at[b],
                )
                return carry

            lax.fori_loop(0, CHUNK // L, fire, 0, unroll=True)

        def drain_gathers(c, b):
            pltpu.make_async_copy(
                out_hbm.at[pl.ds(0, CHUNK)], rows_v.at[b], sem_g.at[b]
            ).wait()

        def wait_outcopy(b):
            pltpu.make_async_copy(
                rows_v.at[b], out_hbm.at[pl.ds(0, CHUNK)], sem_o.at[b]
            ).wait()

        def compute(c, b):
            def grp_body(gg, carry):
                g16 = idx_v[pl.ds(c * CHUNK + gg * L, L)]
                m16 = jnp.where(g16 != 0, SCALE, 0.0).astype(jnp.float32)
                for r in range(L):
                    m = m16.at[jnp.full((L,), r, jnp.int32)].get(
                        mode="promise_in_bounds")
                    row = gg * L + r
                    for kk in range(D // L):
                        v = rows_v[b, row, pl.ds(kk * L, L)]
                        rows_v[b, row, pl.ds(kk * L, L)] = v * m
                return carry

            lax.fori_loop(0, CHUNK // L, grp_body, 0, unroll=False)

        for c in range(NBUF - 1):
            stage_and_fire(c, c)

        def outer_body(g, carry):
            for b in range(NBUF):
                c = g * NBUF + b
                drain_gathers(c, b)
                compute(c, b)
                pltpu.async_copy(
                    rows_v.at[b],
                    out_hbm.at[pl.ds(base + c * CHUNK, CHUNK)],
                    sem_o.at[b],
                )
                bp = (b + NBUF - 1) % NBUF

                @pl.when(c + NBUF - 1 < n_chunks)
                def _():
                    @pl.when(c >= 1)
                    def _():
                        wait_outcopy(bp)

                    stage_and_fire(c + NBUF - 1, bp)

            return carry

        lax.fori_loop(0, n_chunks // NBUF, outer_body, 0, unroll=False)

        for c in range(n_chunks - NBUF, n_chunks):
            wait_outcopy(c % NBUF)

    return k(idx_flat, table)


def kernel(inputs, shared_weights):
    B = inputs.size
    idx_flat = inputs.reshape(B).astype(jnp.int32)
    idx_flat = lax.optimization_barrier(idx_flat)
    out = _sc_lookup(idx_flat, shared_weights, B)
    return out.reshape(inputs.shape + (D,))
